# baseline (speedup 1.0000x reference)
import functools

import jax
import jax.numpy as jnp
from jax import lax
from jax.experimental import pallas as pl
from jax.experimental.pallas import tpu as pltpu
from jax.experimental.pallas import tpu_sc as plsc


def _transpose_table(tT):
    Dd, V = tT.shape
    CW = 512
    grid = (V + CW - 1) // CW

    def body(t_ref, o_ref):
        r = lax.broadcasted_iota(jnp.int32, (Dd, Dd), 0)
        c = lax.broadcasted_iota(jnp.int32, (Dd, Dd), 1)
        eye = jnp.where(r == c, 1.0, 0.0).astype(jnp.float32)
        o_ref[...] = lax.dot_general(
            t_ref[...], eye, (((0,), (0,)), ((), ())),
            preferred_element_type=jnp.float32,
        )

    return pl.pallas_call(
        body,
        grid=(grid,),
        in_specs=[pl.BlockSpec((Dd, CW), lambda i: (0, i))],
        out_specs=pl.BlockSpec((CW, Dd), lambda i: (i, 0)),
        out_shape=jax.ShapeDtypeStruct((V, Dd), jnp.float32),
    )(tT)


def _make_gather(S, B0, V, D):
    info = plsc.get_sparse_core_info()
    nc, ns = info.num_cores, info.num_subcores
    nw = nc * ns
    bw = B0 // nw
    L = info.num_lanes
    ng = bw // L
    pack = 128 // D

    mesh = plsc.VectorSubcoreMesh(core_axis_name="c", subcore_axis_name="s")

    @functools.partial(
        pl.kernel,
        mesh=mesh,
        compiler_params=pltpu.CompilerParams(
            use_tc_tiling_on_sc=True, needs_layout_passes=False
        ),
        out_type=jax.ShapeDtypeStruct((S, D, B0), jnp.float32),
        scratch_types=[
            pltpu.VMEM((56, bw), jnp.int32),
            pltpu.VMEM((2, bw), jnp.int32),
            pltpu.VMEM((2, bw), jnp.int32),
            pltpu.VMEM((2, bw, 128), jnp.float32),
            pltpu.VMEM((2, D, bw), jnp.float32),
            [pltpu.SemaphoreType.DMA] * 2,
            [pltpu.SemaphoreType.DMA] * 2,
        ],
    )
    def gather_kernel(xt_hbm, t4_hbm, out_hbm, idx_v, line_v, off_v,
                      gath_v, block_v, gsems, osems):
        wid = lax.axis_index("s") * nc + lax.axis_index("c")
        b0 = wid * bw
        pltpu.sync_copy(xt_hbm.at[:, pl.ds(b0, bw)], idx_v.at[pl.ds(0, S)])

        def prep_and_fire(s, buf):
            def per_group(g, _):
                iv = idx_v[s, pl.ds(g * L, L)]
                line_v[buf, pl.ds(g * L, L)] = lax.shift_right_logical(iv, 2)
                off_v[buf, pl.ds(g * L, L)] = (iv & (pack - 1)) * D
                return _

            lax.fori_loop(0, ng, per_group, None)
            pltpu.async_copy(
                t4_hbm.at[line_v.at[buf]], gath_v.at[buf], gsems[buf]
            )

        def wait_gather(buf):
            pltpu.make_async_copy(
                t4_hbm.at[pl.ds(0, bw), :], gath_v.at[buf], gsems[buf]
            ).wait()

        def transpose_block(buf):
            rows = lax.broadcasted_iota(jnp.int32, (L,), 0)

            def per_cell(t, _):
                d = t // ng
                g = lax.rem(t, ng)
                offs = off_v[buf, pl.ds(g * L, L)]
                vals = plsc.load_gather(
                    gath_v.at[buf], [rows + g * L, offs + d]
                )
                block_v[buf, d, pl.ds(g * L, L)] = vals
                return _

            lax.fori_loop(0, D * ng, per_cell, None)

        def store_block(s, buf):
            pltpu.async_copy(
                block_v.at[buf], out_hbm.at[s, :, pl.ds(b0, bw)], osems[buf]
            )

        def wait_store(buf):
            pltpu.make_async_copy(
                block_v.at[buf], out_hbm.at[0, :, pl.ds(b0, bw)], osems[buf]
            ).wait()

        prep_and_fire(0, 0)

        def step(s, buf, nbuf):
            @pl.when(s + 1 < S)
            def _fire_next():
                prep_and_fire(s + 1, nbuf)

            wait_gather(buf)

            @pl.when(s >= 2)
            def _drain_store():
                wait_store(buf)

            transpose_block(buf)
            store_block(s, buf)

        def per_pair(k, _):
            step(2 * k, 0, 1)
            step(2 * k + 1, 1, 0)
            return _

        lax.fori_loop(0, S // 2, per_pair, None)
        wait_store(0)
        wait_store(1)

    return gather_kernel


def kernel(x, table):
    B0, S = x.shape
    V, D = table.shape
    t_rm = _transpose_table(table.T)
    t4 = t_rm.reshape(V * D // 128, 128)
    outT = _make_gather(S, B0, V, D)(x.T, t4)
    return outT.transpose(2, 0, 1)


# speedup vs baseline: 2.0174x; 2.0174x over previous
import functools

import jax
import jax.numpy as jnp
from jax import lax
from jax.experimental import pallas as pl
from jax.experimental.pallas import tpu as pltpu
from jax.experimental.pallas import tpu_sc as plsc


def _transpose_table(tT):
    Dd, V = tT.shape
    CW = 4096
    grid = (V + CW - 1) // CW

    def body(t_ref, o_ref):
        r = lax.broadcasted_iota(jnp.int32, (Dd, Dd), 0)
        c = lax.broadcasted_iota(jnp.int32, (Dd, Dd), 1)
        eye = jnp.where(r == c, 1.0, 0.0).astype(jnp.float32)
        o_ref[...] = lax.dot_general(
            t_ref[...], eye, (((0,), (0,)), ((), ())),
            preferred_element_type=jnp.float32,
        )

    return pl.pallas_call(
        body,
        grid=(grid,),
        in_specs=[pl.BlockSpec((Dd, CW), lambda i: (0, i))],
        out_specs=pl.BlockSpec((CW, Dd), lambda i: (i, 0)),
        out_shape=jax.ShapeDtypeStruct((V, Dd), jnp.float32),
    )(tT)


def _make_gather(S, B0, V, D):
    info = plsc.get_sparse_core_info()
    nc, ns = info.num_cores, info.num_subcores
    nw = nc * ns
    bw = B0 // nw
    L = info.num_lanes
    ng = bw // L
    pack = 128 // D

    mesh = plsc.VectorSubcoreMesh(core_axis_name="c", subcore_axis_name="s")

    @functools.partial(
        pl.kernel,
        mesh=mesh,
        compiler_params=pltpu.CompilerParams(
            use_tc_tiling_on_sc=True, needs_layout_passes=False
        ),
        out_type=jax.ShapeDtypeStruct((S, D, B0), jnp.float32),
        scratch_types=[
            pltpu.VMEM((56, bw), jnp.int32),
            pltpu.VMEM((2, bw), jnp.int32),
            pltpu.VMEM((2, bw), jnp.int32),
            pltpu.VMEM((2, bw, 128), jnp.float32),
            pltpu.VMEM((2, D, bw), jnp.float32),
            [pltpu.SemaphoreType.DMA] * 2,
            [pltpu.SemaphoreType.DMA] * 2,
        ],
    )
    def gather_kernel(xt_hbm, t4_hbm, out_hbm, idx_v, line_v, off_v,
                      gath_v, block_v, gsems, osems):
        wid = lax.axis_index("s") * nc + lax.axis_index("c")
        b0 = wid * bw
        pltpu.sync_copy(xt_hbm.at[:, pl.ds(b0, bw)], idx_v.at[pl.ds(0, S)])

        def prep_and_fire(s, buf):
            def per_group(g, _):
                iv = idx_v[s, pl.ds(g * L, L)]
                line_v[buf, pl.ds(g * L, L)] = lax.shift_right_logical(iv, 2)
                off_v[buf, pl.ds(g * L, L)] = (iv & (pack - 1)) * D
                return _

            lax.fori_loop(0, ng, per_group, None)
            pltpu.async_copy(
                t4_hbm.at[line_v.at[buf]], gath_v.at[buf], gsems[buf]
            )

        def wait_gather(buf):
            pltpu.make_async_copy(
                t4_hbm.at[pl.ds(0, bw), :], gath_v.at[buf], gsems[buf]
            ).wait()

        def transpose_block(buf):
            rows = lax.broadcasted_iota(jnp.int32, (L,), 0)

            def per_cell(t, _):
                d = t // ng
                g = lax.rem(t, ng)
                offs = off_v[buf, pl.ds(g * L, L)]
                vals = plsc.load_gather(
                    gath_v.at[buf], [rows + g * L, offs + d]
                )
                block_v[buf, d, pl.ds(g * L, L)] = vals
                return _

            lax.fori_loop(0, D * ng, per_cell, None)

        def store_block(s, buf):
            pltpu.async_copy(
                block_v.at[buf], out_hbm.at[s, :, pl.ds(b0, bw)], osems[buf]
            )

        def wait_store(buf):
            pltpu.make_async_copy(
                block_v.at[buf], out_hbm.at[0, :, pl.ds(b0, bw)], osems[buf]
            ).wait()

        prep_and_fire(0, 0)

        def step(s, buf, nbuf):
            @pl.when(s + 1 < S)
            def _fire_next():
                prep_and_fire(s + 1, nbuf)

            wait_gather(buf)

            @pl.when(s >= 2)
            def _drain_store():
                wait_store(buf)

            transpose_block(buf)
            store_block(s, buf)

        def per_pair(k, _):
            step(2 * k, 0, 1)
            step(2 * k + 1, 1, 0)
            return _

        lax.fori_loop(0, S // 2, per_pair, None)
        wait_store(0)
        wait_store(1)

    return gather_kernel


def kernel(x, table):
    B0, S = x.shape
    V, D = table.shape
    t_rm = _transpose_table(table.T)
    t4 = t_rm.reshape(V * D // 128, 128)
    outT = _make_gather(S, B0, V, D)(x.T, t4)
    return outT.transpose(2, 0, 1)


# speedup vs baseline: 2.2925x; 1.1363x over previous
import functools

import jax
import jax.numpy as jnp
from jax import lax
from jax.experimental import pallas as pl
from jax.experimental.pallas import tpu as pltpu
from jax.experimental.pallas import tpu_sc as plsc


def _transpose_table(tT):
    Dd, V = tT.shape
    CW = 16384
    grid = (V + CW - 1) // CW

    def body(t_ref, o_ref):
        r = lax.broadcasted_iota(jnp.int32, (Dd, Dd), 0)
        c = lax.broadcasted_iota(jnp.int32, (Dd, Dd), 1)
        eye = jnp.where(r == c, 1.0, 0.0).astype(jnp.float32)
        o_ref[...] = lax.dot_general(
            t_ref[...], eye, (((0,), (0,)), ((), ())),
            preferred_element_type=jnp.float32,
        )

    return pl.pallas_call(
        body,
        grid=(grid,),
        in_specs=[pl.BlockSpec((Dd, CW), lambda i: (0, i))],
        out_specs=pl.BlockSpec((CW, Dd), lambda i: (i, 0)),
        out_shape=jax.ShapeDtypeStruct((V, Dd), jnp.float32),
    )(tT)


def _make_gather(S, B0, V, D):
    info = plsc.get_sparse_core_info()
    nc, ns = info.num_cores, info.num_subcores
    nw = nc * ns
    bw = B0 // nw
    L = info.num_lanes
    ng = bw // L
    pack = 128 // D

    mesh = plsc.VectorSubcoreMesh(core_axis_name="c", subcore_axis_name="s")

    @functools.partial(
        pl.kernel,
        mesh=mesh,
        compiler_params=pltpu.CompilerParams(
            use_tc_tiling_on_sc=True, needs_layout_passes=False
        ),
        out_type=jax.ShapeDtypeStruct((S, D, B0), jnp.float32),
        scratch_types=[
            pltpu.VMEM((56, bw), jnp.int32),
            pltpu.VMEM((2, bw), jnp.int32),
            pltpu.VMEM((2, bw), jnp.int32),
            pltpu.VMEM((2, bw, 128), jnp.float32),
            pltpu.VMEM((2, D, bw), jnp.float32),
            [pltpu.SemaphoreType.DMA] * 2,
            [pltpu.SemaphoreType.DMA] * 2,
        ],
    )
    def gather_kernel(xt_hbm, t4_hbm, out_hbm, idx_v, line_v, off_v,
                      gath_v, block_v, gsems, osems):
        wid = lax.axis_index("s") * nc + lax.axis_index("c")
        b0 = wid * bw
        pltpu.sync_copy(xt_hbm.at[:, pl.ds(b0, bw)], idx_v.at[pl.ds(0, S)])

        def prep_and_fire(s, buf):
            def per_group(g, _):
                iv = idx_v[s, pl.ds(g * L, L)]
                line_v[buf, pl.ds(g * L, L)] = lax.shift_right_logical(iv, 2)
                off_v[buf, pl.ds(g * L, L)] = (iv & (pack - 1)) * D
                return _

            lax.fori_loop(0, ng, per_group, None)
            pltpu.async_copy(
                t4_hbm.at[line_v.at[buf]], gath_v.at[buf], gsems[buf]
            )

        def wait_gather(buf):
            pltpu.make_async_copy(
                t4_hbm.at[pl.ds(0, bw), :], gath_v.at[buf], gsems[buf]
            ).wait()

        def transpose_block(buf):
            rows = lax.broadcasted_iota(jnp.int32, (L,), 0)

            def per_cell(t, _):
                d = t // ng
                g = lax.rem(t, ng)
                offs = off_v[buf, pl.ds(g * L, L)]
                vals = plsc.load_gather(
                    gath_v.at[buf], [rows + g * L, offs + d]
                )
                block_v[buf, d, pl.ds(g * L, L)] = vals
                return _

            lax.fori_loop(0, D * ng, per_cell, None)

        def store_block(s, buf):
            pltpu.async_copy(
                block_v.at[buf], out_hbm.at[s, :, pl.ds(b0, bw)], osems[buf]
            )

        def wait_store(buf):
            pltpu.make_async_copy(
                block_v.at[buf], out_hbm.at[0, :, pl.ds(b0, bw)], osems[buf]
            ).wait()

        prep_and_fire(0, 0)

        def step(s, buf, nbuf):
            @pl.when(s + 1 < S)
            def _fire_next():
                prep_and_fire(s + 1, nbuf)

            wait_gather(buf)

            @pl.when(s >= 2)
            def _drain_store():
                wait_store(buf)

            transpose_block(buf)
            store_block(s, buf)

        def per_pair(k, _):
            step(2 * k, 0, 1)
            step(2 * k + 1, 1, 0)
            return _

        lax.fori_loop(0, S // 2, per_pair, None)
        wait_store(0)
        wait_store(1)

    return gather_kernel


def kernel(x, table):
    B0, S = x.shape
    V, D = table.shape
    t_rm = _transpose_table(table.T)
    t4 = t_rm.reshape(V * D // 128, 128)
    outT = _make_gather(S, B0, V, D)(x.T, t4)
    return outT.transpose(2, 0, 1)


# speedup vs baseline: 2.6767x; 1.1676x over previous
import functools

import jax
import jax.numpy as jnp
from jax import lax
from jax.experimental import pallas as pl
from jax.experimental.pallas import tpu as pltpu
from jax.experimental.pallas import tpu_sc as plsc


def _make_gather(S, B0, V, D, SP):
    info = plsc.get_sparse_core_info()
    nc, ns = info.num_cores, info.num_subcores
    nw = nc * ns
    bw = B0 // nw
    L = info.num_lanes
    ng = bw // L

    mesh = plsc.VectorSubcoreMesh(core_axis_name="c", subcore_axis_name="s")

    @functools.partial(
        pl.kernel,
        mesh=mesh,
        compiler_params=pltpu.CompilerParams(
            use_tc_tiling_on_sc=False, needs_layout_passes=False
        ),
        out_type=jax.ShapeDtypeStruct((S, D, B0), jnp.float32),
        scratch_types=[
            pltpu.VMEM((S, bw), jnp.int32),
            pltpu.VMEM((2, bw, D), jnp.float32),
            pltpu.VMEM((2, D, bw), jnp.float32),
            [pltpu.SemaphoreType.DMA] * 2,
            [pltpu.SemaphoreType.DMA] * 2,
        ],
    )
    def gather_kernel(xp_hbm, t_hbm, out_hbm, idx_v, gath_v, block_v,
                      gsems, osems):
        wid = lax.axis_index("s") * nc + lax.axis_index("c")
        b0 = wid * bw
        pltpu.sync_copy(xp_hbm.at[pl.ds(0, S), pl.ds(b0, bw)], idx_v)

        def fire(s, buf):
            pltpu.async_copy(
                t_hbm.at[idx_v.at[s]], gath_v.at[buf], gsems[buf]
            )

        def wait_gather(buf):
            pltpu.make_async_copy(
                t_hbm.at[pl.ds(0, bw), :], gath_v.at[buf], gsems[buf]
            ).wait()

        rows = lax.broadcasted_iota(jnp.int32, (L,), 0)

        def transpose_block(buf):
            def per_d(d, _):
                dv = rows * 0 + d
                for g in range(ng):
                    vals = plsc.load_gather(
                        gath_v.at[buf], [rows + g * L, dv]
                    )
                    block_v[buf, d, pl.ds(g * L, L)] = vals
                return _

            lax.fori_loop(0, D, per_d, None)

        def store_block(s, buf):
            pltpu.async_copy(
                block_v.at[buf], out_hbm.at[s, :, pl.ds(b0, bw)], osems[buf]
            )

        def wait_store(buf):
            pltpu.make_async_copy(
                block_v.at[buf], out_hbm.at[0, :, pl.ds(b0, bw)], osems[buf]
            ).wait()

        fire(0, 0)

        def step(s, buf, nbuf):
            @pl.when(s + 1 < S)
            def _fire_next():
                fire(s + 1, nbuf)

            wait_gather(buf)

            @pl.when(s >= 2)
            def _drain_store():
                wait_store(buf)

            transpose_block(buf)
            store_block(s, buf)

        def per_pair(k, _):
            step(2 * k, 0, 1)
            step(2 * k + 1, 1, 0)
            return _

        lax.fori_loop(0, S // 2, per_pair, None)
        wait_store(0)
        wait_store(1)

    return gather_kernel


def kernel(x, table):
    B0, S = x.shape
    V, D = table.shape
    t_rm = table
    SP = 56
    xp = jnp.pad(x.T, ((0, SP - S), (0, 0)))
    outT = _make_gather(S, B0, V, D, SP)(xp, t_rm)
    return outT.transpose(2, 0, 1)


# speedup vs baseline: 2.6782x; 1.0006x over previous
import functools

import jax
import jax.numpy as jnp
from jax import lax
from jax.experimental import pallas as pl
from jax.experimental.pallas import tpu as pltpu
from jax.experimental.pallas import tpu_sc as plsc


def _make_gather(S, B0, V, D, SP):
    info = plsc.get_sparse_core_info()
    nc, ns = info.num_cores, info.num_subcores
    nw = nc * ns
    bw = B0 // nw
    L = info.num_lanes
    ng = bw // L

    mesh = plsc.VectorSubcoreMesh(core_axis_name="c", subcore_axis_name="s")

    @functools.partial(
        pl.kernel,
        mesh=mesh,
        compiler_params=pltpu.CompilerParams(
            use_tc_tiling_on_sc=False, needs_layout_passes=False
        ),
        out_type=jax.ShapeDtypeStruct((S, D, B0), jnp.float32),
        scratch_types=[
            pltpu.VMEM((S, bw), jnp.int32),
            pltpu.VMEM((4, bw, D), jnp.float32),
            pltpu.VMEM((2, D, bw), jnp.float32),
            [pltpu.SemaphoreType.DMA] * 4,
            [pltpu.SemaphoreType.DMA] * 2,
        ],
    )
    def gather_kernel(xp_hbm, t_hbm, out_hbm, idx_v, gath_v, block_v,
                      gsems, osems):
        wid = lax.axis_index("s") * nc + lax.axis_index("c")
        b0 = wid * bw
        pltpu.sync_copy(xp_hbm.at[pl.ds(0, S), pl.ds(b0, bw)], idx_v)

        def fire(s, buf):
            pltpu.async_copy(
                t_hbm.at[idx_v.at[s]], gath_v.at[buf], gsems[buf]
            )

        def wait_gather(buf):
            pltpu.make_async_copy(
                t_hbm.at[pl.ds(0, bw), :], gath_v.at[buf], gsems[buf]
            ).wait()

        rows = lax.broadcasted_iota(jnp.int32, (L,), 0)

        def transpose_block(gbuf, bbuf):
            def per_d(d, _):
                dv = rows * 0 + d
                for g in range(ng):
                    vals = plsc.load_gather(
                        gath_v.at[gbuf], [rows + g * L, dv]
                    )
                    block_v[bbuf, d, pl.ds(g * L, L)] = vals
                return _

            lax.fori_loop(0, D, per_d, None)

        def store_block(s, buf):
            pltpu.async_copy(
                block_v.at[buf], out_hbm.at[s, :, pl.ds(b0, bw)], osems[buf]
            )

        def wait_store(buf):
            pltpu.make_async_copy(
                block_v.at[buf], out_hbm.at[0, :, pl.ds(b0, bw)], osems[buf]
            ).wait()

        fire(0, 0)
        fire(1, 1)
        fire(2, 2)

        def step(s, gbuf, bbuf):
            @pl.when(s + 3 < S)
            def _fire_ahead():
                fire(s + 3, (gbuf + 3) % 4)

            wait_gather(gbuf)

            @pl.when(s >= 2)
            def _drain_store():
                wait_store(bbuf)

            transpose_block(gbuf, bbuf)
            store_block(s, bbuf)

        def per_quad(q, _):
            s = 4 * q
            step(s, 0, 0)
            step(s + 1, 1, 1)
            step(s + 2, 2, 0)
            step(s + 3, 3, 1)
            return _

        lax.fori_loop(0, S // 4, per_quad, None)
        step(S - 2, 0, 0)
        step(S - 1, 1, 1)
        wait_store(0)
        wait_store(1)

    return gather_kernel


def kernel(x, table):
    B0, S = x.shape
    V, D = table.shape
    t_rm = table
    SP = 56
    xp = jnp.pad(x.T, ((0, SP - S), (0, 0)))
    outT = _make_gather(S, B0, V, D, SP)(xp, t_rm)
    return outT.transpose(2, 0, 1)
